# Initial kernel scaffold; baseline (speedup 1.0000x reference)
import jax
import jax.numpy as jnp
from jax.experimental import pallas as pl


def kernel(pred_boxes, pred_scores, pred_labels, target_boxes, target_labels):
    raise NotImplementedError("write your pallas kernel here")


# speedup vs baseline: 1.0563x; 1.0563x over previous
import functools

import jax
import jax.numpy as jnp
from jax.experimental import pallas as pl

_NP = 20000
_NPP = 20096
_NT = 1000
_TB = 8
_NBLK = _NT // _TB


def _body(tref, pref, iou_ref, lab_ref):
    t = tref[0]
    ttl = [t[:, d:d + 1] for d in range(3)]
    tbr = [t[:, 3 + d:4 + d] for d in range(3)]
    ptl = [pref[d:d + 1, :] for d in range(3)]
    pbr = [pref[3 + d:4 + d, :] for d in range(3)]
    score = pref[6:7, :]
    label = pref[7:8, :]

    vt = ((tbr[0] - ttl[0] + 1.0) * (tbr[1] - ttl[1] + 1.0)
          * (tbr[2] - ttl[2] + 1.0))
    vp = ((pbr[0] - ptl[0] + 1.0) * (pbr[1] - ptl[1] + 1.0)
          * (pbr[2] - ptl[2] + 1.0))

    e0 = jnp.minimum(tbr[0], pbr[0]) - jnp.maximum(ttl[0], ptl[0]) + 1.0
    e1 = jnp.minimum(tbr[1], pbr[1]) - jnp.maximum(ttl[1], ptl[1]) + 1.0
    e2 = jnp.minimum(tbr[2], pbr[2]) - jnp.maximum(ttl[2], ptl[2]) + 1.0
    inter = (e0 * e1) * e2
    union = (vt + vp) - inter

    ov1 = (tbr[0] > ptl[0]) | (tbr[1] > ptl[1]) | (tbr[2] > ptl[2])
    ov2 = (ttl[0] < pbr[0]) | (ttl[1] < pbr[1]) | (ttl[2] < pbr[2])
    valid = score > 0.5
    keep = (ov1 & ov2) & valid

    iou = inter / jnp.where(keep, union, 1.0)
    val = jnp.where(keep, iou, 0.0)

    jvec = jax.lax.broadcasted_iota(jnp.int32, (1, _NPP), 1)
    real = jvec < _NP
    val = jnp.where(real, val, -jnp.inf)
    key = jnp.where(valid, score, jnp.inf)

    bm = jnp.max(val, axis=1, keepdims=True)
    tie = val == bm
    km = jnp.min(jnp.where(tie, key, jnp.inf), axis=1, keepdims=True)
    tie2 = tie & (key == km)
    jm = jnp.min(jnp.where(tie2, jvec, jnp.int32(2 ** 30)), axis=1,
                 keepdims=True)
    tie3 = tie2 & (jvec == jm)
    lab = jnp.max(jnp.where(tie3, label, -jnp.inf), axis=1, keepdims=True)

    iou_ref[0] = bm
    lab_ref[0] = lab


@jax.jit
def _run(tgt, preds):
    out = pl.pallas_call(
        _body,
        grid=(_NBLK,),
        in_specs=[
            pl.BlockSpec((1, _TB, 8), lambda i: (i, 0, 0)),
            pl.BlockSpec((8, _NPP), lambda i: (0, 0)),
        ],
        out_specs=[
            pl.BlockSpec((1, _TB, 1), lambda i: (i, 0, 0)),
            pl.BlockSpec((1, _TB, 1), lambda i: (i, 0, 0)),
        ],
        out_shape=[
            jax.ShapeDtypeStruct((_NBLK, _TB, 1), jnp.float32),
            jax.ShapeDtypeStruct((_NBLK, _TB, 1), jnp.float32),
        ],
    )(tgt, preds)
    return out


def kernel(pred_boxes, pred_scores, pred_labels, target_boxes, target_labels):
    preds = jnp.concatenate(
        [pred_boxes, pred_scores[:, None], pred_labels[:, None]], axis=1).T
    preds = jnp.pad(preds, ((0, 0), (0, _NPP - _NP)))
    tgt = jnp.concatenate(
        [target_boxes, jnp.zeros((_NT, 2), jnp.float32)], axis=1)
    tgt = tgt.reshape(_NBLK, _TB, 8)
    iou3, lab3 = _run(tgt, preds)
    true_ious = iou3.reshape(_NT)
    pcp_best = lab3.reshape(_NT)
    hit = true_ious > 0.5
    return true_ious, pcp_best, hit, target_labels


# speedup vs baseline: 1.4312x; 1.3548x over previous
import functools

import jax
import jax.numpy as jnp
from jax.experimental import pallas as pl
from jax.experimental.pallas import tpu as pltpu

_NP = 20000
_NPP = 20096
_NT = 1000
_TB = 8
_NBLK = _NT // _TB


def _body(tref, pref, iou_ref, lab_ref, scr):
    @pl.when(pl.program_id(0) == 0)
    def _init():
        score = pref[6]
        scr[0:8, :] = jnp.where(score > 0.5, score, jnp.inf)
        vp = (((pref[3] - pref[0] + 1.0) * (pref[4] - pref[1] + 1.0))
              * (pref[5] - pref[2] + 1.0))
        scr[8:16, :] = jnp.where(score > 0.5, vp, jnp.inf)
        scr[16:24, :] = jax.lax.broadcasted_iota(
            jnp.int32, (8, _NPP), 1).astype(jnp.float32)

    t = tref[0]
    ttl = [t[:, d:d + 1] for d in range(3)]
    tbr = [t[:, 3 + d:4 + d] for d in range(3)]
    vt = ((tbr[0] - ttl[0] + 1.0) * (tbr[1] - ttl[1] + 1.0)
          * (tbr[2] - ttl[2] + 1.0))

    key = scr[0:8, :]
    vp = scr[8:16, :]
    jvec = scr[16:24, :]
    label = pref[7]

    e0 = jnp.minimum(tbr[0], pref[3]) - jnp.maximum(ttl[0], pref[0]) + 1.0
    e1 = jnp.minimum(tbr[1], pref[4]) - jnp.maximum(ttl[1], pref[1]) + 1.0
    e2 = jnp.minimum(tbr[2], pref[5]) - jnp.maximum(ttl[2], pref[2]) + 1.0
    inter = (e0 * e1) * e2
    union = (vt + vp) - inter

    ov1 = (tbr[0] > pref[0]) | (tbr[1] > pref[1]) | (tbr[2] > pref[2])
    ov2 = (ttl[0] < pref[3]) | (ttl[1] < pref[4]) | (ttl[2] < pref[5])
    keep = ov1 & ov2

    val = jnp.where(keep, inter / union, 0.0)

    bm = jnp.max(val, axis=1, keepdims=True)
    tie = val == bm
    km = jnp.min(jnp.where(tie, key, jnp.inf), axis=1, keepdims=True)
    tie2 = tie & (key == km)
    jm = jnp.min(jnp.where(tie2, jvec, jnp.float32(2 ** 30)), axis=1,
                 keepdims=True)
    tie3 = tie2 & (jvec == jm)
    lab = jnp.max(jnp.where(tie3, label, -jnp.inf), axis=1, keepdims=True)

    iou_ref[0] = bm
    lab_ref[0] = lab


@jax.jit
def _run(tgt, preds):
    out = pl.pallas_call(
        _body,
        grid=(_NBLK,),
        in_specs=[
            pl.BlockSpec((1, _TB, 8), lambda i: (i, 0, 0)),
            pl.BlockSpec((8, 8, _NPP), lambda i: (0, 0, 0)),
        ],
        out_specs=[
            pl.BlockSpec((1, _TB, 1), lambda i: (i, 0, 0)),
            pl.BlockSpec((1, _TB, 1), lambda i: (i, 0, 0)),
        ],
        out_shape=[
            jax.ShapeDtypeStruct((_NBLK, _TB, 1), jnp.float32),
            jax.ShapeDtypeStruct((_NBLK, _TB, 1), jnp.float32),
        ],
        scratch_shapes=[pltpu.VMEM((24, _NPP), jnp.float32)],
    )(tgt, preds)
    return out


def kernel(pred_boxes, pred_scores, pred_labels, target_boxes, target_labels):
    preds = jnp.concatenate(
        [pred_boxes, pred_scores[:, None], pred_labels[:, None]], axis=1).T
    preds = jnp.pad(preds, ((0, 0), (0, _NPP - _NP)))
    preds = jnp.broadcast_to(preds[:, None, :], (8, 8, _NPP))
    tgt = jnp.concatenate(
        [target_boxes, jnp.zeros((_NT, 2), jnp.float32)], axis=1)
    tgt = tgt.reshape(_NBLK, _TB, 8)
    iou3, lab3 = _run(tgt, preds)
    true_ious = iou3.reshape(_NT)
    pcp_best = lab3.reshape(_NT)
    hit = true_ious > 0.5
    return true_ious, pcp_best, hit, target_labels
